# baseline (speedup 1.0000x reference)
import functools

import jax
import jax.numpy as jnp
from jax import lax
from jax.experimental import pallas as pl
from jax.experimental.pallas import tpu as pltpu
from jax.experimental.pallas import tpu_sc as plsc

D_MODEL = 64
NUM_CORES = 2
NUM_SUBCORES = 16
NUM_WORKERS = NUM_CORES * NUM_SUBCORES
CHUNK = 128


@functools.lru_cache(maxsize=None)
def _make_lookup(nchunk: int, d: int):
  mesh = plsc.VectorSubcoreMesh(
      core_axis_name="c", subcore_axis_name="s",
      num_cores=NUM_CORES, num_subcores=NUM_SUBCORES)

  @functools.partial(
      pl.kernel,
      out_type=jax.ShapeDtypeStruct((NUM_WORKERS, nchunk, CHUNK, d),
                                    jnp.float32),
      mesh=mesh,
      compiler_params=pltpu.CompilerParams(use_tc_tiling_on_sc=False),
      scratch_types=[
          pltpu.VMEM((nchunk, CHUNK), jnp.int32),
          pltpu.VMEM((CHUNK, d), jnp.float32),
          pltpu.SemaphoreType.DMA,
          pltpu.SemaphoreType.DMA,
      ],
  )
  def lookup(idx_hbm, table_hbm, out_hbm, idx_v, rows_v, sem_i, sem_g):
    wid = lax.axis_index("s") * NUM_CORES + lax.axis_index("c")
    pltpu.async_copy(idx_hbm.at[wid], idx_v, sem_i).wait()

    def chunk_body(j, carry):
      pltpu.async_copy(table_hbm.at[idx_v.at[j]], rows_v, sem_g).wait()
      pltpu.sync_copy(rows_v, out_hbm.at[wid, j])
      return carry

    lax.fori_loop(0, nchunk, chunk_body, None)

  return lookup


def kernel(x, embedding_weight):
  b, l = x.shape
  n = b * l
  nchunk = n // (NUM_WORKERS * CHUNK)
  assert nchunk * NUM_WORKERS * CHUNK == n
  idx = x.reshape(NUM_WORKERS, nchunk, CHUNK).astype(jnp.int32)
  d = embedding_weight.shape[1]
  out = _make_lookup(nchunk, d)(idx, embedding_weight)
  return out.reshape(b, l, d)


# speedup vs baseline: 1.1118x; 1.1118x over previous
import functools

import jax
import jax.numpy as jnp
from jax import lax
from jax.experimental import pallas as pl
from jax.experimental.pallas import tpu as pltpu
from jax.experimental.pallas import tpu_sc as plsc

D_MODEL = 64
NUM_CORES = 2
NUM_SUBCORES = 16
NUM_WORKERS = NUM_CORES * NUM_SUBCORES
CHUNK = 128
GK = 5


@functools.lru_cache(maxsize=None)
def _make_lookup(nchunk: int, d: int):
  assert nchunk % (2 * GK) == 0
  ngroups = nchunk // GK
  mesh = plsc.VectorSubcoreMesh(
      core_axis_name="c", subcore_axis_name="s",
      num_cores=NUM_CORES, num_subcores=NUM_SUBCORES)

  @functools.partial(
      pl.kernel,
      out_type=jax.ShapeDtypeStruct((NUM_WORKERS, nchunk, CHUNK, d),
                                    jnp.float32),
      mesh=mesh,
      compiler_params=pltpu.CompilerParams(use_tc_tiling_on_sc=False),
      scratch_types=[
          pltpu.VMEM((nchunk, CHUNK), jnp.int32),
          pltpu.VMEM((GK, CHUNK, d), jnp.float32),
          pltpu.VMEM((GK, CHUNK, d), jnp.float32),
          pltpu.SemaphoreType.DMA,
          pltpu.SemaphoreType.DMA,
          pltpu.SemaphoreType.DMA,
          pltpu.SemaphoreType.DMA,
          pltpu.SemaphoreType.DMA,
      ],
  )
  def lookup(idx_hbm, table_hbm, out_hbm, idx_v, rows0, rows1,
             sem_i, sem_g0, sem_g1, sem_o0, sem_o1):
    wid = lax.axis_index("s") * NUM_CORES + lax.axis_index("c")
    rows = (rows0, rows1)
    sem_g = (sem_g0, sem_g1)
    sem_o = (sem_o0, sem_o1)

    def fire_gathers(t, cur):
      for b in range(GK):
        pltpu.async_copy(table_hbm.at[idx_v.at[t * GK + b]],
                         rows[cur].at[b], sem_g[cur])

    def drain_gathers(t, cur):
      for b in range(GK):
        pltpu.make_async_copy(table_hbm.at[idx_v.at[t * GK + b]],
                              rows[cur].at[b], sem_g[cur]).wait()

    def start_out(t, cur):
      pltpu.async_copy(rows[cur], out_hbm.at[wid, pl.ds(t * GK, GK)],
                       sem_o[cur])

    def wait_out(t, cur):
      pltpu.make_async_copy(rows[cur], out_hbm.at[wid, pl.ds(t * GK, GK)],
                            sem_o[cur]).wait()

    pltpu.async_copy(idx_hbm.at[wid], idx_v, sem_i).wait()

    fire_gathers(0, 0)
    fire_gathers(1, 1)
    drain_gathers(0, 0)
    start_out(0, 0)

    def pair_body(p, carry):
      for cur in range(2):
        t = 2 * p + cur
        wait_out(t - 2, cur)
        fire_gathers(t, cur)
        drain_gathers(t - 1, 1 - cur)
        start_out(t - 1, 1 - cur)
      return carry

    lax.fori_loop(1, ngroups // 2, pair_body, None)

    drain_gathers(ngroups - 1, 1)
    start_out(ngroups - 1, 1)
    wait_out(ngroups - 2, 0)
    wait_out(ngroups - 1, 1)

  return lookup


def kernel(x, embedding_weight):
  b, l = x.shape
  n = b * l
  nchunk = n // (NUM_WORKERS * CHUNK)
  assert nchunk * NUM_WORKERS * CHUNK == n
  idx = x.reshape(NUM_WORKERS, nchunk, CHUNK).astype(jnp.int32)
  d = embedding_weight.shape[1]
  out = _make_lookup(nchunk, d)(idx, embedding_weight)
  return out.reshape(b, l, d)
